# baseline (speedup 1.0000x reference)
import jax
import jax.numpy as jnp
from jax import lax
from jax.experimental import pallas as pl
from jax.experimental.pallas import tpu as pltpu
from jax.experimental.pallas import tpu_sc as plsc

_L = 16385
_D = 64
_NREL = 1000
_CHUNK = 64
_NW = 32
_CPW = 8
_ROWS_PW = _CHUNK * _CPW
_MAIN = _NW * _ROWS_PW
_PPAD = _MAIN + 16


def _sc_body(path_hbm, table_hbm, out_hbm, pbuf, cidx, rows, tidx, trows,
             sem_g, sem_w):
    nc = 2
    wid = lax.axis_index("s") * nc + lax.axis_index("c")
    off = (lax.iota(jnp.int32, 16) & 1) * _NREL

    base = wid * _ROWS_PW
    pltpu.sync_copy(path_hbm.at[pl.ds(base, _ROWS_PW)], pbuf)
    gathers = []
    for j in range(_CPW):
        cj = cidx.at[j]
        for k in range(_CHUNK // 16):
            cj[pl.ds(k * 16, 16)] = pbuf[pl.ds(j * _CHUNK + k * 16, 16)] + off
        gathers.append(
            pltpu.async_copy(
                table_hbm.at[cj],
                rows.at[pl.ds(j * _CHUNK, _CHUNK)],
                sem_g,
            )
        )
    writes = []
    for j in range(_CPW):
        gathers[j].wait()
        writes.append(
            pltpu.async_copy(
                rows.at[pl.ds(j * _CHUNK, _CHUNK)],
                out_hbm.at[pl.ds(base + j * _CHUNK, _CHUNK)],
                sem_w,
            )
        )

    @pl.when(wid == 0)
    def _():
        pltpu.sync_copy(path_hbm.at[pl.ds(_MAIN - 16, 16)], tidx)
        pltpu.sync_copy(path_hbm.at[pl.ds(_MAIN, 1)], tidx.at[pl.ds(0, 1)])
        tidx[...] = tidx[...] + off
        pltpu.async_copy(table_hbm.at[tidx], trows, sem_g).wait()
        pltpu.async_copy(
            trows.at[pl.ds(0, 1)], out_hbm.at[pl.ds(_MAIN, 1)], sem_w
        ).wait()

    for w in writes:
        w.wait()


def kernel(path, W_ent, W_rel):
    table = jnp.concatenate([W_ent[:_NREL], W_rel[:_NREL]], axis=0)
    p = path.astype(jnp.int32)
    mesh = plsc.VectorSubcoreMesh(core_axis_name="c", subcore_axis_name="s")
    out = pl.kernel(
        _sc_body,
        mesh=mesh,
        compiler_params=pltpu.CompilerParams(use_tc_tiling_on_sc=False),
        out_type=jax.ShapeDtypeStruct((_L, _D), jnp.float32),
        scratch_types=[
            pltpu.VMEM((_ROWS_PW,), jnp.int32),
            pltpu.VMEM((_CPW, _CHUNK), jnp.int32),
            pltpu.VMEM((_ROWS_PW, _D), jnp.float32),
            pltpu.VMEM((16,), jnp.int32),
            pltpu.VMEM((16, _D), jnp.float32),
            pltpu.SemaphoreType.DMA,
            pltpu.SemaphoreType.DMA,
        ],
    )(p, table)
    return out
